# baseline (speedup 1.0000x reference)
import functools
import math

import jax
import jax.numpy as jnp
from jax import lax
from jax.experimental import pallas as pl
from jax.experimental.pallas import tpu as pltpu
from jax.experimental.pallas import tpu_sc as plsc

DT_SCALE = math.log1p(168.0)

B, L = 16384, 200
NC, NS = 2, 16
NW = NC * NS
CW = B // NW
RBLK = 8
N_CHUNKS = L // RBLK
NBUF = 6

_mesh = plsc.VectorSubcoreMesh(core_axis_name="c", subcore_axis_name="s")


@functools.partial(
    pl.kernel,
    mesh=_mesh,
    compiler_params=pltpu.CompilerParams(needs_layout_passes=False),
    out_type=jax.ShapeDtypeStruct((L, B), jnp.float32),
    scratch_types=[
        pltpu.VMEM((16,), jnp.float32),
        pltpu.VMEM((16,), jnp.float32),
        pltpu.VMEM((NBUF, RBLK, CW), jnp.float32),
        pltpu.VMEM((NBUF, RBLK, CW), jnp.float32),
        pltpu.VMEM((NBUF, RBLK, CW), jnp.int32),
        pltpu.VMEM((NBUF, RBLK, CW), jnp.float32),
        pltpu.SemaphoreType.DMA((NBUF,)),
        pltpu.SemaphoreType.DMA((NBUF,)),
    ],
)
def _sc_recovery(mpc_hbm, dt_hbm, idx_hbm, ltau_hbm, out_hbm,
                 ltau_v, itau_v, mpc_v, dt_v, idx_v, out_v,
                 in_sem, out_sem):
    wid = lax.axis_index("s") * NC + lax.axis_index("c")
    col0 = wid * CW

    def in_copies(c, b):
        rs = pl.ds(c * RBLK, RBLK)
        cs = pl.ds(col0, CW)
        return (
            pltpu.make_async_copy(mpc_hbm.at[rs, cs], mpc_v.at[b], in_sem.at[b]),
            pltpu.make_async_copy(dt_hbm.at[rs, cs], dt_v.at[b], in_sem.at[b]),
            pltpu.make_async_copy(idx_hbm.at[rs, cs], idx_v.at[b], in_sem.at[b]),
        )

    def out_copy(c, b):
        return pltpu.make_async_copy(
            out_v.at[b], out_hbm.at[pl.ds(c * RBLK, RBLK), pl.ds(col0, CW)],
            out_sem.at[b])

    def start_in(c, b):
        for cp in in_copies(c, b):
            cp.start()

    for c in range(NBUF):
        start_in(c, c)

    pltpu.sync_copy(ltau_hbm, ltau_v.at[pl.ds(0, 15)])
    itau_v[...] = -jnp.exp(-ltau_v[...])

    def chunk_body(c, carry):
        b = c % NBUF
        for cp in in_copies(c, b):
            cp.wait()

        @pl.when(c >= NBUF)
        def _drain():
            out_copy(c - NBUF, b).wait()

        @plsc.parallel_loop(0, RBLK * CW, step=16, unroll=4)
        def _compute(i):
            r = i >> 9
            col = i & (CW - 1)
            s = pl.ds(col, 16)
            neg_inv_tau = plsc.load_gather(itau_v, [idx_v[b, r, s]])
            e1 = jnp.exp(dt_v[b, r, s] * DT_SCALE)
            out_v[b, r, s] = 1.0 - (1.0 - mpc_v[b, r, s]) * jnp.exp(
                (e1 - 1.0) * neg_inv_tau)

        out_copy(c, b).start()

        @pl.when(c + NBUF < N_CHUNKS)
        def _prefetch():
            start_in(c + NBUF, b)

        return carry

    lax.fori_loop(0, N_CHUNKS, chunk_body, 0)

    for c in range(N_CHUNKS - NBUF, N_CHUNKS):
        out_copy(c, c % NBUF).wait()


def kernel(mpc, delta_t, muscle_idx, log_tau):
    out_t = _sc_recovery(mpc.T, delta_t.T, muscle_idx.astype(jnp.int32).T,
                         log_tau)
    return out_t.T


# speedup vs baseline: 1.0256x; 1.0256x over previous
import functools
import math

import jax
import jax.numpy as jnp
from jax import lax
from jax.experimental import pallas as pl
from jax.experimental.pallas import tpu as pltpu
from jax.experimental.pallas import tpu_sc as plsc

DT_SCALE = math.log1p(168.0)

B, L = 16384, 200
NC, NS = 2, 16
NW = NC * NS
CW = B // NW
RBLK = 8
N_CHUNKS = L // RBLK
NBUF = 4

_mesh = plsc.VectorSubcoreMesh(core_axis_name="c", subcore_axis_name="s")


@functools.partial(
    pl.kernel,
    mesh=_mesh,
    compiler_params=pltpu.CompilerParams(needs_layout_passes=False),
    out_type=jax.ShapeDtypeStruct((L, B), jnp.float32),
    scratch_types=[
        pltpu.VMEM((16,), jnp.float32),
        pltpu.VMEM((16,), jnp.float32),
        pltpu.VMEM((NBUF, RBLK, CW), jnp.float32),
        pltpu.VMEM((NBUF, RBLK, CW), jnp.float32),
        pltpu.VMEM((NBUF, RBLK, CW), jnp.int32),
        pltpu.VMEM((NBUF, RBLK, CW), jnp.float32),
        pltpu.SemaphoreType.DMA((NBUF,)),
        pltpu.SemaphoreType.DMA((NBUF,)),
    ],
)
def _sc_recovery(mpc_hbm, dt_hbm, idx_hbm, ltau_hbm, out_hbm,
                 ltau_v, itau_v, mpc_v, dt_v, idx_v, out_v,
                 in_sem, out_sem):
    wid = lax.axis_index("s") * NC + lax.axis_index("c")
    col0 = wid * CW

    def in_copies(c, b):
        rs = pl.ds(c * RBLK, RBLK)
        cs = pl.ds(col0, CW)
        return (
            pltpu.make_async_copy(mpc_hbm.at[rs, cs], mpc_v.at[b], in_sem.at[b]),
            pltpu.make_async_copy(dt_hbm.at[rs, cs], dt_v.at[b], in_sem.at[b]),
            pltpu.make_async_copy(idx_hbm.at[rs, cs], idx_v.at[b], in_sem.at[b]),
        )

    def out_copy(c, b):
        return pltpu.make_async_copy(
            out_v.at[b], out_hbm.at[pl.ds(c * RBLK, RBLK), pl.ds(col0, CW)],
            out_sem.at[b])

    def start_in(c, b):
        for cp in in_copies(c, b):
            cp.start()

    for c in range(NBUF):
        start_in(c, c)

    pltpu.sync_copy(ltau_hbm, ltau_v.at[pl.ds(0, 15)])
    itau_v[...] = -jnp.exp(-ltau_v[...])

    def chunk_body(c, carry):
        b = c & (NBUF - 1)
        for cp in in_copies(c, b):
            cp.wait()

        @pl.when(c >= NBUF)
        def _drain():
            out_copy(c - NBUF, b).wait()

        @plsc.parallel_loop(0, RBLK * CW, step=16, unroll=8)
        def _compute(i):
            r = i >> 9
            col = i & (CW - 1)
            s = pl.ds(col, 16)
            neg_inv_tau = plsc.load_gather(itau_v, [idx_v[b, r, s]])
            e1 = jnp.exp(dt_v[b, r, s] * DT_SCALE)
            out_v[b, r, s] = 1.0 - (1.0 - mpc_v[b, r, s]) * jnp.exp(
                (e1 - 1.0) * neg_inv_tau)

        out_copy(c, b).start()

        @pl.when(c + NBUF < N_CHUNKS)
        def _prefetch():
            start_in(c + NBUF, b)

        return carry

    lax.fori_loop(0, N_CHUNKS, chunk_body, 0)

    for c in range(N_CHUNKS - NBUF, N_CHUNKS):
        out_copy(c, c & (NBUF - 1)).wait()


def kernel(mpc, delta_t, muscle_idx, log_tau):
    out_t = _sc_recovery(mpc.T, delta_t.T, muscle_idx.astype(jnp.int32).T,
                         log_tau)
    return out_t.T
